# baseline (speedup 1.0000x reference)
import functools

import jax
import jax.numpy as jnp
from jax.experimental import pallas as pl
from jax.experimental.pallas import tpu as pltpu

T, B, C, H, W = 4, 4, 192, 16, 16
E, TOPK, HID, HEADS = 8, 2, 768, 8
N = H * W
RT = B * N
R = T * RT
D = C // HEADS
F32 = jnp.float32
BF16 = jnp.bfloat16


def _split_hi_lo(w):
    hi = w.astype(BF16)
    lo = (w - hi.astype(F32)).astype(BF16)
    return hi, lo


def _dot2(s_bf, w_hi, w_lo):
    return (jnp.dot(s_bf, w_hi, preferred_element_type=F32)
            + jnp.dot(s_bf, w_lo, preferred_element_type=F32))


def _bn_rows(y):
    m = jnp.mean(y, axis=0, keepdims=True)
    c = y - m
    v = jnp.mean(c * c, axis=0, keepdims=True)
    return c * jax.lax.rsqrt(v + 1e-5)


def _lif4(y, tau):
    rows = y.shape[0] // T
    v = jnp.zeros((rows, y.shape[1]), F32)
    outs = []
    for t in range(T):
        xt = y[t * rows:(t + 1) * rows, :]
        v = v + (xt - v) / tau
        s = (v >= 1.0).astype(F32)
        v = v * (1.0 - s)
        outs.append(s)
    return jnp.concatenate(outs, axis=0)


def _ssa_router_kernel(x_ref, wq_hi, wq_lo, wk_hi, wk_lo, wv_hi, wv_lo,
                       wp_hi, wp_lo, wrt_ref,
                       h_ref, wfull_ref, q_s, k_s, v_s, o_s):
    x = x_ref[...]
    s = _lif4(x, 2.0).astype(BF16)
    dot = functools.partial(jnp.dot, preferred_element_type=F32)
    q_s[...] = _lif4(_bn_rows(_dot2(s, wq_hi[...], wq_lo[...])), 2.0).astype(BF16)
    k_s[...] = _lif4(_bn_rows(_dot2(s, wk_hi[...], wk_lo[...])), 2.0).astype(BF16)
    v_s[...] = _lif4(_bn_rows(_dot2(s, wv_hi[...], wv_lo[...])), 2.0).astype(BF16)

    def tb_body(i, carry):
        base = i * N
        q_tb = q_s[pl.ds(base, N), :]
        k_tb = k_s[pl.ds(base, N), :]
        v_tb = v_s[pl.ds(base, N), :]
        parts = []
        for hh in range(HEADS):
            sl = slice(hh * D, (hh + 1) * D)
            a = jax.lax.dot_general(
                q_tb[:, sl], k_tb[:, sl],
                (((1,), (1,)), ((), ())),
                preferred_element_type=F32) * 0.125
            parts.append(dot(a.astype(BF16), v_tb[:, sl]))
        o_s[pl.ds(base, N), :] = jnp.concatenate(parts, axis=1)
        return carry

    jax.lax.fori_loop(0, T * B, tb_body, 0)

    o_sp = _lif4(o_s[...], 2.0).astype(BF16)
    h = x + _bn_rows(_dot2(o_sp, wp_hi[...], wp_lo[...]))
    h_ref[...] = h

    xm = 0.25 * (h[0:RT, :] + h[RT:2 * RT, :] + h[2 * RT:3 * RT, :]
                 + h[3 * RT:4 * RT, :])
    r = dot(xm, wrt_ref[...])
    rb = _bn_rows(r)
    logits = jnp.concatenate(
        [jnp.mean(rb[b * N:(b + 1) * N, :], axis=0, keepdims=True)
         for b in range(B)], axis=0)
    mx = jnp.max(logits, axis=-1, keepdims=True)
    ex = jnp.exp(logits - mx)
    probs = ex / jnp.sum(ex, axis=-1, keepdims=True)
    iota = jax.lax.broadcasted_iota(jnp.int32, (B, E), 1)
    m1 = jnp.max(probs, axis=-1, keepdims=True)
    i1 = jnp.min(jnp.where(probs == m1, iota, E), axis=-1, keepdims=True)
    oh1 = iota == i1
    pmasked = jnp.where(oh1, -1.0, probs)
    m2 = jnp.max(pmasked, axis=-1, keepdims=True)
    i2 = jnp.min(jnp.where(pmasked == m2, iota, E), axis=-1, keepdims=True)
    oh2 = iota == i2
    p1 = jnp.sum(jnp.where(oh1, probs, 0.0), axis=-1, keepdims=True)
    p2 = jnp.sum(jnp.where(oh2, probs, 0.0), axis=-1, keepdims=True)
    tot = p1 + p2
    wfull_ref[...] = jnp.where(oh1, p1 / tot, 0.0) + jnp.where(oh2, p2 / tot, 0.0)


def _experts_kernel(taus_ref, wfull_ref, h_ref, w1_hi, w1_lo, w2_hi, w2_lo,
                    out_ref):
    e = pl.program_id(0)

    @pl.when(e == 0)
    def _init():
        out_ref[...] = h_ref[...]

    wb = [wfull_ref[b, e] for b in range(B)]
    selected = (wb[0] > 0) | (wb[1] > 0) | (wb[2] > 0) | (wb[3] > 0)

    @pl.when(selected)
    def _compute():
        tau = taus_ref[0, e]
        h = h_ref[...]
        s = _lif4(h, tau).astype(BF16)
        y1 = _bn_rows(_dot2(s, w1_hi[0], w1_lo[0]))
        s2 = _lif4(y1, tau).astype(BF16)
        yb = _bn_rows(_dot2(s2, w2_hi[0], w2_lo[0]))
        for t in range(T):
            for b in range(B):
                lo = t * RT + b * N
                sl = slice(lo, lo + N)
                out_ref[sl, :] += wb[b] * yb[sl, :]


def kernel(x, Wq, Wk, Wv, Wp, Wr, W1, W2):
    x_r = x.reshape(T, B, C, N).transpose(0, 1, 3, 2).reshape(R, C)
    taus = jnp.linspace(1.5, 4.0, E, dtype=F32).reshape(1, E)
    wq_hi, wq_lo = _split_hi_lo(Wq.T)
    wk_hi, wk_lo = _split_hi_lo(Wk.T)
    wv_hi, wv_lo = _split_hi_lo(Wv.T)
    wp_hi, wp_lo = _split_hi_lo(Wp.T)
    w1_hi, w1_lo = _split_hi_lo(W1.transpose(0, 2, 1))
    w2_hi, w2_lo = _split_hi_lo(W2.transpose(0, 2, 1))

    h, wfull = pl.pallas_call(
        _ssa_router_kernel,
        out_shape=[jax.ShapeDtypeStruct((R, C), F32),
                   jax.ShapeDtypeStruct((B, E), F32)],
        scratch_shapes=[pltpu.VMEM((R, C), BF16)] * 3
        + [pltpu.VMEM((R, C), F32)],
    )(x_r, wq_hi, wq_lo, wk_hi, wk_lo, wv_hi, wv_lo, wp_hi, wp_lo, Wr.T)

    wspec = lambda sh: pl.BlockSpec(sh, lambda e: (e, 0, 0))
    out = pl.pallas_call(
        _experts_kernel,
        grid=(E,),
        in_specs=[
            pl.BlockSpec(memory_space=pltpu.SMEM),
            pl.BlockSpec(memory_space=pltpu.SMEM),
            pl.BlockSpec((R, C), lambda e: (0, 0)),
            wspec((1, C, HID)), wspec((1, C, HID)),
            wspec((1, HID, C)), wspec((1, HID, C)),
        ],
        out_specs=pl.BlockSpec((R, C), lambda e: (0, 0)),
        out_shape=jax.ShapeDtypeStruct((R, C), F32),
    )(taus, wfull, h, w1_hi, w1_lo, w2_hi, w2_lo)

    return out.reshape(T, B, N, C).transpose(0, 1, 3, 2).reshape(T, B, C, H, W)


# speedup vs baseline: 1.7105x; 1.7105x over previous
import jax
import jax.numpy as jnp
from jax.experimental import pallas as pl
from jax.experimental.pallas import tpu as pltpu

T, B, C, H, W = 4, 4, 192, 16, 16
E, TOPK, HID, HEADS = 8, 2, 768, 8
N = H * W
RT = B * N
R = T * RT
D = C // HEADS
F32 = jnp.float32


def _bn_scale_shift(y):
    m = jnp.sum(y, axis=0, keepdims=True) * (1.0 / R)
    msq = jnp.sum(y * y, axis=0, keepdims=True) * (1.0 / R)
    g = jax.lax.rsqrt(msq - m * m + 1e-5)
    return g, -m * g


def _lif4(y, tau, g=None, sh=None):
    r = 1.0 / tau
    a = r if g is None else r * g
    b = None if g is None else r * sh
    one_m_r = 1.0 - r
    rows = y.shape[0] // T
    v = 0.0
    outs = []
    for t in range(T):
        yt = y[t * rows:(t + 1) * rows, :]
        u = yt * a if b is None else yt * a + b
        v = u if t == 0 else v * one_m_r + u
        fire = v >= 1.0
        outs.append(jnp.where(fire, 1.0, 0.0))
        v = jnp.where(fire, 0.0, v)
    return jnp.concatenate(outs, axis=0)


def _ssa_router_kernel(x_ref, wqt_ref, wkt_ref, wvt_ref, wpt_ref, wrt_ref,
                       h_ref, wfull_ref, q_s, k_s, v_s, o_s):
    x = x_ref[...]
    s = _lif4(x, 2.0)

    def dot(aa, bb):
        return jnp.dot(aa, bb, preferred_element_type=F32)

    yq = dot(s, wqt_ref[...])
    yk = dot(s, wkt_ref[...])
    yv = dot(s, wvt_ref[...])
    q_s[...] = _lif4(yq, 2.0, *_bn_scale_shift(yq))
    k_s[...] = _lif4(yk, 2.0, *_bn_scale_shift(yk))
    v_s[...] = _lif4(yv, 2.0, *_bn_scale_shift(yv))

    def tb_body(i, carry):
        base = i * N
        q_tb = q_s[pl.ds(base, N), :]
        k_tb = k_s[pl.ds(base, N), :]
        v_tb = v_s[pl.ds(base, N), :]
        parts = []
        for hh in range(HEADS):
            sl = slice(hh * D, (hh + 1) * D)
            a = jax.lax.dot_general(
                q_tb[:, sl], k_tb[:, sl],
                (((1,), (1,)), ((), ())),
                preferred_element_type=F32) * 0.125
            parts.append(dot(a, v_tb[:, sl]))
        o_s[pl.ds(base, N), :] = jnp.concatenate(parts, axis=1)
        return carry

    jax.lax.fori_loop(0, T * B, tb_body, 0)

    o_sp = _lif4(o_s[...], 2.0)
    yp = dot(o_sp, wpt_ref[...])
    g, sh = _bn_scale_shift(yp)
    h = x + (yp * g + sh)
    h_ref[...] = h

    xm = 0.25 * (h[0:RT, :] + h[RT:2 * RT, :] + h[2 * RT:3 * RT, :]
                 + h[3 * RT:4 * RT, :])
    rr = dot(xm, wrt_ref[...])
    mr = jnp.sum(rr, axis=0, keepdims=True) * (1.0 / RT)
    vr = jnp.sum(rr * rr, axis=0, keepdims=True) * (1.0 / RT) - mr * mr
    gr = jax.lax.rsqrt(vr + 1e-5)
    logits = jnp.concatenate(
        [jnp.sum(rr[b * N:(b + 1) * N, :], axis=0, keepdims=True) * (1.0 / N)
         for b in range(B)], axis=0)
    logits = (logits - mr) * gr
    mx = jnp.max(logits, axis=-1, keepdims=True)
    ex = jnp.exp(logits - mx)
    probs = ex / jnp.sum(ex, axis=-1, keepdims=True)
    iota = jax.lax.broadcasted_iota(jnp.int32, (B, E), 1)
    m1 = jnp.max(probs, axis=-1, keepdims=True)
    i1 = jnp.min(jnp.where(probs == m1, iota, E), axis=-1, keepdims=True)
    oh1 = iota == i1
    pmasked = jnp.where(oh1, -1.0, probs)
    m2 = jnp.max(pmasked, axis=-1, keepdims=True)
    i2 = jnp.min(jnp.where(pmasked == m2, iota, E), axis=-1, keepdims=True)
    oh2 = iota == i2
    p1 = jnp.sum(jnp.where(oh1, probs, 0.0), axis=-1, keepdims=True)
    p2 = jnp.sum(jnp.where(oh2, probs, 0.0), axis=-1, keepdims=True)
    tot = p1 + p2
    wfull_ref[...] = jnp.where(oh1, p1 / tot, 0.0) + jnp.where(oh2, p2 / tot, 0.0)


def _experts_kernel(taus_ref, wfull_ref, h_ref, w1t_ref, w2t_ref, out_ref):
    e = pl.program_id(0)

    @pl.when(e == 0)
    def _init():
        out_ref[...] = h_ref[...]

    wb = [wfull_ref[b, e] for b in range(B)]
    selected = (wb[0] > 0) | (wb[1] > 0) | (wb[2] > 0) | (wb[3] > 0)

    @pl.when(selected)
    def _compute():
        tau = taus_ref[0, e]
        h = h_ref[...]
        s = _lif4(h, tau)
        y1 = jnp.dot(s, w1t_ref[0], preferred_element_type=F32)
        s2 = _lif4(y1, tau, *_bn_scale_shift(y1))
        y2 = jnp.dot(s2, w2t_ref[0], preferred_element_type=F32)
        g, sh = _bn_scale_shift(y2)
        for t in range(T):
            for b in range(B):
                lo = t * RT + b * N
                sl = slice(lo, lo + N)
                out_ref[sl, :] += y2[sl, :] * (wb[b] * g) + wb[b] * sh


def kernel(x, Wq, Wk, Wv, Wp, Wr, W1, W2):
    x_r = x.reshape(T, B, C, N).transpose(0, 1, 3, 2).reshape(R, C)
    taus = jnp.linspace(1.5, 4.0, E, dtype=F32).reshape(1, E)

    h, wfull = pl.pallas_call(
        _ssa_router_kernel,
        out_shape=[jax.ShapeDtypeStruct((R, C), F32),
                   jax.ShapeDtypeStruct((B, E), F32)],
        scratch_shapes=[pltpu.VMEM((R, C), F32)] * 4,
    )(x_r, Wq.T, Wk.T, Wv.T, Wp.T, Wr.T)

    out = pl.pallas_call(
        _experts_kernel,
        grid=(E,),
        in_specs=[
            pl.BlockSpec(memory_space=pltpu.SMEM),
            pl.BlockSpec(memory_space=pltpu.SMEM),
            pl.BlockSpec((R, C), lambda e: (0, 0)),
            pl.BlockSpec((1, C, HID), lambda e: (e, 0, 0)),
            pl.BlockSpec((1, HID, C), lambda e: (e, 0, 0)),
        ],
        out_specs=pl.BlockSpec((R, C), lambda e: (0, 0)),
        out_shape=jax.ShapeDtypeStruct((R, C), F32),
    )(taus, wfull, h, W1.transpose(0, 2, 1), W2.transpose(0, 2, 1))

    return out.reshape(T, B, N, C).transpose(0, 1, 3, 2).reshape(T, B, C, H, W)


# speedup vs baseline: 1.9235x; 1.1246x over previous
import jax
import jax.numpy as jnp
from jax.experimental import pallas as pl
from jax.experimental.pallas import tpu as pltpu

T, B, C, H, W = 4, 4, 192, 16, 16
E, TOPK, HID, HEADS = 8, 2, 768, 8
N = H * W
RT = B * N
R = T * RT
D = C // HEADS
QKV = 3 * C
F32 = jnp.float32


def _dott(a, b):
    return jax.lax.dot_general(a, b, (((1,), (1,)), ((), ())),
                               preferred_element_type=F32)


def _bn_scale_shift(y):
    m = jnp.sum(y, axis=0, keepdims=True) * (1.0 / R)
    msq = jnp.sum(y * y, axis=0, keepdims=True) * (1.0 / R)
    g = jax.lax.rsqrt(msq - m * m + 1e-5)
    return g, -m * g


def _lif_store(dst, y, tau, g=None, sh=None):
    r = 1.0 / tau
    c = 1.0 - r
    rows = y.shape[0] // T
    if g is None:
        theta, rho, crho = tau, 0.0, None
    else:
        a = r * g
        theta = (1.0 - sh) / a
        rho = sh / (-a)
        crho = c * rho
    vt = None
    for t in range(T):
        yt = y[t * rows:(t + 1) * rows, :]
        if t == 0:
            vt = yt if crho is None else crho + yt
        else:
            vt = c * vt + yt
        fire = vt >= theta
        dst[pl.ds(t * rows, rows), :] = jnp.where(fire, 1.0, 0.0)
        vt = jnp.where(fire, rho, vt)


def _ssa_router_kernel(x_ref, wqkv_ref, wp_ref, wr_ref,
                       h_ref, wfull_ref, s_s, qkv_s, o_s):
    x = x_ref[...]
    _lif_store(s_s, x, 2.0)
    y_qkv = _dott(s_s[...], wqkv_ref[...])
    _lif_store(qkv_s, y_qkv, 2.0, *_bn_scale_shift(y_qkv))

    def tb_body(i, carry):
        base = i * N
        qkv_tb = qkv_s[pl.ds(base, N), :]
        parts = []
        for hh in range(HEADS):
            qsl = slice(hh * D, (hh + 1) * D)
            ksl = slice(C + hh * D, C + (hh + 1) * D)
            vsl = slice(2 * C + hh * D, 2 * C + (hh + 1) * D)
            a = _dott(qkv_tb[:, qsl], qkv_tb[:, ksl]) * 0.125
            parts.append(jnp.dot(a, qkv_tb[:, vsl],
                                 preferred_element_type=F32))
        o_s[pl.ds(base, N), :] = jnp.concatenate(parts, axis=1)
        return carry

    jax.lax.fori_loop(0, T * B, tb_body, 0)

    _lif_store(s_s, o_s[...], 2.0)
    yp = _dott(s_s[...], wp_ref[...])
    g, sh = _bn_scale_shift(yp)
    h = x + (yp * g + sh)
    h_ref[...] = h

    xm = 0.25 * (h[0:RT, :] + h[RT:2 * RT, :] + h[2 * RT:3 * RT, :]
                 + h[3 * RT:4 * RT, :])
    rr = _dott(xm, wr_ref[...])
    mr = jnp.sum(rr, axis=0, keepdims=True) * (1.0 / RT)
    vr = jnp.sum(rr * rr, axis=0, keepdims=True) * (1.0 / RT) - mr * mr
    gr = jax.lax.rsqrt(vr + 1e-5)
    logits = jnp.concatenate(
        [jnp.sum(rr[b * N:(b + 1) * N, :], axis=0, keepdims=True) * (1.0 / N)
         for b in range(B)], axis=0)
    logits = (logits - mr) * gr
    mx = jnp.max(logits, axis=-1, keepdims=True)
    ex = jnp.exp(logits - mx)
    probs = ex / jnp.sum(ex, axis=-1, keepdims=True)
    iota = jax.lax.broadcasted_iota(jnp.int32, (B, E), 1)
    m1 = jnp.max(probs, axis=-1, keepdims=True)
    i1 = jnp.min(jnp.where(probs == m1, iota, E), axis=-1, keepdims=True)
    oh1 = iota == i1
    pmasked = jnp.where(oh1, -1.0, probs)
    m2 = jnp.max(pmasked, axis=-1, keepdims=True)
    i2 = jnp.min(jnp.where(pmasked == m2, iota, E), axis=-1, keepdims=True)
    oh2 = iota == i2
    p1 = jnp.sum(jnp.where(oh1, probs, 0.0), axis=-1, keepdims=True)
    p2 = jnp.sum(jnp.where(oh2, probs, 0.0), axis=-1, keepdims=True)
    tot = p1 + p2
    wfull_ref[...] = jnp.where(oh1, p1 / tot, 0.0) + jnp.where(oh2, p2 / tot, 0.0)


def _experts_kernel(taus_ref, wfull_ref, h_ref, w1_ref, w2_ref, out_ref,
                    s_s, s2_s):
    e = pl.program_id(0)

    @pl.when(e == 0)
    def _init():
        out_ref[...] = h_ref[...]

    wb = [wfull_ref[b, e] for b in range(B)]
    selected = (wb[0] > 0) | (wb[1] > 0) | (wb[2] > 0) | (wb[3] > 0)

    @pl.when(selected)
    def _compute():
        tau = taus_ref[0, e]
        _lif_store(s_s, h_ref[...], tau)
        y1 = _dott(s_s[...], w1_ref[0])
        _lif_store(s2_s, y1, tau, *_bn_scale_shift(y1))
        y2 = _dott(s2_s[...], w2_ref[0])
        g, sh = _bn_scale_shift(y2)
        for t in range(T):
            for b in range(B):
                lo = t * RT + b * N
                sl = slice(lo, lo + N)
                out_ref[sl, :] += y2[sl, :] * (wb[b] * g) + wb[b] * sh


def kernel(x, Wq, Wk, Wv, Wp, Wr, W1, W2):
    x_r = x.reshape(T, B, C, N).transpose(0, 1, 3, 2).reshape(R, C)
    taus = jnp.linspace(1.5, 4.0, E, dtype=F32).reshape(1, E)
    wqkv = jnp.concatenate([Wq, Wk, Wv], axis=0)

    h, wfull = pl.pallas_call(
        _ssa_router_kernel,
        out_shape=[jax.ShapeDtypeStruct((R, C), F32),
                   jax.ShapeDtypeStruct((B, E), F32)],
        scratch_shapes=[pltpu.VMEM((R, C), F32),
                        pltpu.VMEM((R, QKV), F32),
                        pltpu.VMEM((R, C), F32)],
    )(x_r, wqkv, Wp, Wr)

    out = pl.pallas_call(
        _experts_kernel,
        grid=(E,),
        in_specs=[
            pl.BlockSpec(memory_space=pltpu.SMEM),
            pl.BlockSpec(memory_space=pltpu.SMEM),
            pl.BlockSpec((R, C), lambda e: (0, 0)),
            pl.BlockSpec((1, HID, C), lambda e: (e, 0, 0)),
            pl.BlockSpec((1, C, HID), lambda e: (e, 0, 0)),
        ],
        out_specs=pl.BlockSpec((R, C), lambda e: (0, 0)),
        out_shape=jax.ShapeDtypeStruct((R, C), F32),
        scratch_shapes=[pltpu.VMEM((R, C), F32),
                        pltpu.VMEM((R, HID), F32)],
    )(taus, wfull, h, W1, W2)

    return out.reshape(T, B, N, C).transpose(0, 1, 3, 2).reshape(T, B, C, H, W)
